# baseline (speedup 1.0000x reference)
import functools

import jax
import jax.numpy as jnp
from jax import lax
from jax.experimental import pallas as pl
from jax.experimental.pallas import tpu as pltpu
from jax.experimental.pallas import tpu_sc as plsc

_P = 16
_NW = 32
_NE = 16


def _contract_last(a, b):
    return jax.lax.dot_general(
        a, b, (((1,), (1,)), ((), ())), preferred_element_type=jnp.float32
    )


def _gate_kernel(x_ref, wg_ref, g_ref):
    xb = x_ref[...].astype(jnp.bfloat16)
    wg = wg_ref[...].astype(jnp.bfloat16)
    g_ref[0] = _contract_last(wg, xb)


def _gate_logits(xf, wg, ct):
    t, d = xf.shape
    nch = t // ct
    return pl.pallas_call(
        _gate_kernel,
        grid=(nch,),
        in_specs=[
            pl.BlockSpec((ct, d), lambda i: (i, 0)),
            pl.BlockSpec(wg.shape, lambda i: (0, 0)),
        ],
        out_specs=pl.BlockSpec((1, _NE, ct), lambda i: (i, 0, 0)),
        out_shape=jax.ShapeDtypeStruct((nch, _NE, ct), jnp.float32),
    )(xf, wg)


def _make_scale_kernel(t):
    ct = t // _NW

    @functools.partial(
        pl.kernel,
        mesh=plsc.VectorSubcoreMesh(core_axis_name="c", subcore_axis_name="s"),
        out_type=jax.ShapeDtypeStruct((t,), jnp.float32),
        scratch_types=[
            pltpu.VMEM((_NE, ct), jnp.float32),
            pltpu.VMEM((ct,), jnp.float32),
        ],
    )
    def scale_kernel(g_hbm, out_hbm, g_v, s_v):
        wid = lax.axis_index("s") * 2 + lax.axis_index("c")
        pltpu.sync_copy(g_hbm.at[wid], g_v)
        for i in range(ct // 16):
            sl = pl.ds(i * 16, 16)
            regs = [g_v[e, sl] for e in range(_NE)]
            m = regs[0]
            for e in range(1, _NE):
                m = jnp.maximum(m, regs[e])
            es = [jnp.exp(r - m) for r in regs]
            z = es[0]
            for e in range(1, _NE):
                z = z + es[e]
            cnt = jnp.zeros((16,), jnp.float32)
            mx2 = jnp.zeros((16,), jnp.float32)
            for e in range(_NE):
                is1 = es[e] >= 1.0
                cnt = cnt + jnp.where(is1, 1.0, 0.0)
                mx2 = jnp.maximum(mx2, jnp.where(is1, 0.0, es[e]))
            second = jnp.where(cnt >= 2.0, 1.0, mx2)
            s = (1.0 + second) / z
            s_v[sl] = s / (s + 1e-9)
        pltpu.sync_copy(s_v, out_hbm.at[pl.ds(wid * ct, ct)])

    return scale_kernel


def _mlp_kernel(x_ref, scale_ref, wupf_ref, wdownf_ref, o_ref, wub, wdb):
    i = pl.program_id(0)
    cu = wupf_ref.shape[0]
    cd = wdownf_ref.shape[0]

    @pl.when(i < _P)
    def _cast_weights():
        wub[pl.ds(i * cu, cu), :] = wupf_ref[...].astype(jnp.bfloat16)
        wdb[pl.ds(i * cd, cd), :] = wdownf_ref[...].astype(jnp.bfloat16)

    @pl.when(i >= _P)
    def _compute():
        x = x_ref[...].astype(jnp.bfloat16)
        h = _contract_last(x, wub[...])
        hb = h.astype(jnp.bfloat16)
        hb = hb * jax.nn.sigmoid(hb)
        out = _contract_last(hb, wdb[...])
        o_ref[...] = out * scale_ref[...]


@functools.partial(jax.jit, static_argnames=("tm",))
def _run(xf, wg, wup, wdown, tm):
    t, d = xf.shape
    ed = wup.shape[0]
    cu = ed // _P
    cd = d // _P
    nt = t // tm

    logits = _gate_logits(xf, wg, t // _NW)
    scale = _make_scale_kernel(t)(logits)
    scale = scale.reshape(t, 1)

    return pl.pallas_call(
        _mlp_kernel,
        grid=(_P + nt,),
        in_specs=[
            pl.BlockSpec((tm, d), lambda i: (jnp.maximum(i - _P, 0), 0)),
            pl.BlockSpec((tm, 1), lambda i: (jnp.maximum(i - _P, 0), 0)),
            pl.BlockSpec((cu, d), lambda i: (jnp.minimum(i, _P - 1), 0)),
            pl.BlockSpec((cd, ed), lambda i: (jnp.minimum(i, _P - 1), 0)),
        ],
        out_specs=pl.BlockSpec((tm, d), lambda i: (jnp.maximum(i - _P, 0), 0)),
        out_shape=jax.ShapeDtypeStruct((t, d), jnp.float32),
        scratch_shapes=[
            pltpu.VMEM((ed, d), jnp.bfloat16),
            pltpu.VMEM((d, ed), jnp.bfloat16),
        ],
    )(xf, scale, wup, wdown)


def kernel(x, W_g, W_up, W_down):
    b, s, d = x.shape
    xf = x.reshape(b * s, d)
    out = _run(xf, W_g, W_up, W_down, tm=256)
    return out.reshape(b, s, d)
